# baseline (speedup 1.0000x reference)
import functools

import jax
import jax.numpy as jnp
from jax import lax
from jax.experimental import pallas as pl
from jax.experimental.pallas import tpu as pltpu
from jax.experimental.pallas import tpu_sc as plsc

K = 32
B = 4
S = 4096
T = B * S
LANES = 16
GL = 2 * LANES

NUM_CORES = 2
NUM_SUBCORES = 16
NW = NUM_CORES * NUM_SUBCORES
TOK_PER_W = T // NW
CHUNK = 128
N_CHUNKS = TOK_PER_W // CHUNK
GROUPS = CHUNK // GL


WPB = NW // B
SEQ_PER_W = S // WPB


def _sc_body(vals_hbm, dist_hbm, out_hbm, vals2, dist2, stage2, ones_v,
             in_sems, out_sems):
    wid = lax.axis_index("s") * NUM_CORES + lax.axis_index("c")
    ones_v[...] = jnp.full((LANES,), 0x00010001, jnp.int32)
    bi = wid // WPB
    s_base = (wid % WPB) * SEQ_PER_W
    lane_iota = lax.iota(jnp.int32, LANES)

    def issue_in(ci):
        p = ci % 2
        s0 = s_base + ci * CHUNK
        pltpu.async_copy(
            vals_hbm.at[bi, pl.ds(s0, CHUNK)], vals2.at[p], in_sems.at[p])
        pltpu.async_copy(
            dist_hbm.at[bi, pl.ds(s0, CHUNK)], dist2.at[p], in_sems.at[p])

    def wait_in(ci):
        p = ci % 2
        s0 = s_base + ci * CHUNK
        pltpu.make_async_copy(
            vals_hbm.at[bi, pl.ds(s0, CHUNK)], vals2.at[p],
            in_sems.at[p]).wait()
        pltpu.make_async_copy(
            dist_hbm.at[bi, pl.ds(s0, CHUNK)], dist2.at[p],
            in_sems.at[p]).wait()

    def wait_out(ci):
        p = ci % 2
        s0 = s_base + ci * CHUNK
        pltpu.make_async_copy(
            stage2.at[p], out_hbm.at[bi, pl.ds(s0, CHUNK)],
            out_sems.at[p]).wait()

    issue_in(0)
    for ci in range(N_CHUNKS):
        p = ci % 2
        s0 = s_base + ci * CHUNK
        if ci + 1 < N_CHUNKS:
            issue_in(ci + 1)
        wait_in(ci)
        if ci >= 2:
            wait_out(ci - 2)
        vals_v = vals2.at[p]
        dist_v = dist2.at[p]
        stage_v = stage2.at[p]

        @plsc.parallel_loop(0, GROUPS, unroll=2)
        def group_body(g):
            r0 = g * GL
            rows_lo = r0 + lane_iota
            rows_hi = r0 + LANES + lane_iota
            for t in range(GL):
                for h in range(2):
                    sl = pl.ds(h * LANES, LANES)
                    stage_v[r0 + t, sl] = dist_v[r0 + t, sl]
            cols = []
            for j in range(K):
                cj = jnp.full((LANES,), j, jnp.int32)
                lo = plsc.load_gather(vals_v, [rows_lo, cj])
                hi = plsc.load_gather(vals_v, [rows_hi, cj])
                cols.append(lo | (hi << 16))
            ones16 = plsc.bitcast(ones_v[...], jnp.uint16)
            count = plsc.bitcast(jnp.zeros((LANES,), jnp.int32),
                                 jnp.uint16)
            for j in range(K):
                vj = cols[j]
                terms = [plsc.bitcast(vl ^ vj, jnp.uint16)
                         for vl in cols[:j]]
                terms.append(plsc.bitcast(vj, jnp.uint16))
                while len(terms) > 1:
                    nxt = []
                    for i in range(0, len(terms) - 1, 2):
                        nxt.append(jnp.minimum(terms[i], terms[i + 1]))
                    if len(terms) % 2:
                        nxt.append(terms[-1])
                    terms = nxt
                count = count + jnp.minimum(terms[0], ones16)
                c32 = plsc.bitcast(count, jnp.int32)
                colj = jnp.full((LANES,), K + j, jnp.int32)
                plsc.store_scatter(
                    stage_v, [rows_lo, colj],
                    (c32 & 0xFFFF).astype(jnp.float32))
                plsc.store_scatter(
                    stage_v, [rows_hi, colj],
                    (c32 >> 16).astype(jnp.float32))

        pltpu.async_copy(
            stage_v, out_hbm.at[bi, pl.ds(s0, CHUNK)], out_sems.at[p])

    wait_out(N_CHUNKS - 2)
    wait_out(N_CHUNKS - 1)


@functools.partial(jax.jit, static_argnames=())
def kernel(vals, distances):
    mesh = plsc.VectorSubcoreMesh(
        core_axis_name="c", subcore_axis_name="s",
        num_cores=NUM_CORES, num_subcores=NUM_SUBCORES)
    out = pl.kernel(
        _sc_body,
        out_type=jax.ShapeDtypeStruct((B, S, 2 * K), jnp.float32),
        mesh=mesh,
        scratch_types=[
            pltpu.VMEM((2, CHUNK, K), jnp.int32),
            pltpu.VMEM((2, CHUNK, K), jnp.float32),
            pltpu.VMEM((2, CHUNK, 2 * K), jnp.float32),
            pltpu.VMEM((LANES,), jnp.int32),
            pltpu.SemaphoreType.DMA((2,)),
            pltpu.SemaphoreType.DMA((2,)),
        ],
        compiler_params=pltpu.CompilerParams(needs_layout_passes=False),
    )(vals, distances)
    return out


# speedup vs baseline: 1.0116x; 1.0116x over previous
import functools

import jax
import jax.numpy as jnp
from jax import lax
from jax.experimental import pallas as pl
from jax.experimental.pallas import tpu as pltpu
from jax.experimental.pallas import tpu_sc as plsc

K = 32
B = 4
S = 4096
T = B * S
LANES = 16
GL = 2 * LANES

NUM_CORES = 2
NUM_SUBCORES = 16
NW = NUM_CORES * NUM_SUBCORES
TOK_PER_W = T // NW
CHUNK = 128
N_CHUNKS = TOK_PER_W // CHUNK
GROUPS = CHUNK // GL


WPB = NW // B
SEQ_PER_W = S // WPB


def _sc_body(vals_hbm, dist_hbm, out_hbm, vals2, dist2, stage2, ones_v,
             in_sems, out_sems):
    wid = lax.axis_index("s") * NUM_CORES + lax.axis_index("c")
    ones_v[...] = jnp.full((LANES,), 0x00010001, jnp.int32)
    bi = wid // WPB
    s_base = (wid % WPB) * SEQ_PER_W
    lane_iota = lax.iota(jnp.int32, LANES)

    def issue_in(ci):
        p = ci % 2
        s0 = s_base + ci * CHUNK
        pltpu.async_copy(
            vals_hbm.at[bi, pl.ds(s0, CHUNK)], vals2.at[p], in_sems.at[p])
        pltpu.async_copy(
            dist_hbm.at[bi, pl.ds(s0, CHUNK)], dist2.at[p], in_sems.at[p])

    def wait_in(ci):
        p = ci % 2
        s0 = s_base + ci * CHUNK
        pltpu.make_async_copy(
            vals_hbm.at[bi, pl.ds(s0, CHUNK)], vals2.at[p],
            in_sems.at[p]).wait()
        pltpu.make_async_copy(
            dist_hbm.at[bi, pl.ds(s0, CHUNK)], dist2.at[p],
            in_sems.at[p]).wait()

    def wait_out(ci):
        p = ci % 2
        s0 = s_base + ci * CHUNK
        pltpu.make_async_copy(
            stage2.at[p], out_hbm.at[bi, pl.ds(s0, CHUNK)],
            out_sems.at[p]).wait()

    issue_in(0)
    for ci in range(N_CHUNKS):
        p = ci % 2
        s0 = s_base + ci * CHUNK
        if ci + 1 < N_CHUNKS:
            issue_in(ci + 1)
        wait_in(ci)
        if ci >= 2:
            wait_out(ci - 2)
        vals_v = vals2.at[p]
        dist_v = dist2.at[p]
        stage_v = stage2.at[p]

        @plsc.parallel_loop(0, GROUPS)
        def group_body(g):
            r0 = g * GL
            rows_lo = r0 + lane_iota
            rows_hi = r0 + LANES + lane_iota
            for t in range(GL):
                for h in range(2):
                    sl = pl.ds(h * LANES, LANES)
                    stage_v[r0 + t, sl] = dist_v[r0 + t, sl]
            cols = []
            for j in range(K):
                cj = jnp.full((LANES,), j, jnp.int32)
                lo = plsc.load_gather(vals_v, [rows_lo, cj])
                hi = plsc.load_gather(vals_v, [rows_hi, cj])
                cols.append(lo | (hi << 16))
            ones16 = plsc.bitcast(ones_v[...], jnp.uint16)
            count = plsc.bitcast(jnp.zeros((LANES,), jnp.int32),
                                 jnp.uint16)
            for j in range(K):
                vj = cols[j]
                terms = [plsc.bitcast(vl ^ vj, jnp.uint16)
                         for vl in cols[:j]]
                terms.append(plsc.bitcast(vj, jnp.uint16))
                while len(terms) > 1:
                    nxt = []
                    for i in range(0, len(terms) - 1, 2):
                        nxt.append(jnp.minimum(terms[i], terms[i + 1]))
                    if len(terms) % 2:
                        nxt.append(terms[-1])
                    terms = nxt
                count = count + jnp.minimum(terms[0], ones16)
                c32 = plsc.bitcast(count, jnp.int32)
                colj = jnp.full((LANES,), K + j, jnp.int32)
                plsc.store_scatter(
                    stage_v, [rows_lo, colj],
                    (c32 & 0xFFFF).astype(jnp.float32))
                plsc.store_scatter(
                    stage_v, [rows_hi, colj],
                    (c32 >> 16).astype(jnp.float32))

        pltpu.async_copy(
            stage_v, out_hbm.at[bi, pl.ds(s0, CHUNK)], out_sems.at[p])

    wait_out(N_CHUNKS - 2)
    wait_out(N_CHUNKS - 1)


@functools.partial(jax.jit, static_argnames=())
def kernel(vals, distances):
    mesh = plsc.VectorSubcoreMesh(
        core_axis_name="c", subcore_axis_name="s",
        num_cores=NUM_CORES, num_subcores=NUM_SUBCORES)
    out = pl.kernel(
        _sc_body,
        out_type=jax.ShapeDtypeStruct((B, S, 2 * K), jnp.float32),
        mesh=mesh,
        scratch_types=[
            pltpu.VMEM((2, CHUNK, K), jnp.int32),
            pltpu.VMEM((2, CHUNK, K), jnp.float32),
            pltpu.VMEM((2, CHUNK, 2 * K), jnp.float32),
            pltpu.VMEM((LANES,), jnp.int32),
            pltpu.SemaphoreType.DMA((2,)),
            pltpu.SemaphoreType.DMA((2,)),
        ],
        compiler_params=pltpu.CompilerParams(needs_layout_passes=False),
    )(vals, distances)
    return out
